# Initial kernel scaffold; baseline (speedup 1.0000x reference)
import jax
import jax.numpy as jnp
from jax.experimental import pallas as pl


def kernel(words, weight):
    raise NotImplementedError("write your pallas kernel here")


# speedup vs baseline: 2.7174x; 2.7174x over previous
import functools

import jax
import jax.numpy as jnp
from jax import lax
from jax.experimental import pallas as pl
from jax.experimental.pallas import tpu as pltpu
from jax.experimental.pallas import tpu_sc as plsc

VOCAB = 100000
DIM = 64
EMBED_P = 0.1
BATCH = 4096
HIST = 200

_B = BATCH * HIST

_info = plsc.get_sparse_core_info()
_NC = _info.num_cores
_NS = _info.num_subcores
_NW = _NC * _NS
_BPW = _B // _NW
_CH = 128
_NCHUNK = _BPW // _CH


def _scale_body(w_ref, m_ref, o_ref):
    o_ref[...] = w_ref[...] * m_ref[...]


def _masked_table(weight, mask):
    rows_per_blk = 4000
    grid = VOCAB // rows_per_blk
    return pl.pallas_call(
        _scale_body,
        grid=(grid,),
        in_specs=[
            pl.BlockSpec((rows_per_blk, DIM), lambda i: (i, 0)),
            pl.BlockSpec((rows_per_blk, 1), lambda i: (i, 0)),
        ],
        out_specs=pl.BlockSpec((rows_per_blk, DIM), lambda i: (i, 0)),
        out_shape=jax.ShapeDtypeStruct((VOCAB, DIM), jnp.float32),
    )(weight, mask)


_mesh = plsc.VectorSubcoreMesh(core_axis_name="c", subcore_axis_name="s")


@functools.partial(
    pl.kernel,
    mesh=_mesh,
    out_type=jax.ShapeDtypeStruct((_B, DIM), jnp.float32),
    scratch_types=[
        pltpu.VMEM((_NCHUNK, _CH), jnp.int32),
        pltpu.VMEM((_CH, DIM), jnp.float32),
        pltpu.SemaphoreType.DMA,
    ],
    compiler_params=pltpu.CompilerParams(use_tc_tiling_on_sc=False),
)
def _sc_gather(tab_hbm, idx_hbm, out_hbm, idx_v, rows_v, sem):
    wid = lax.axis_index("s") * _NC + lax.axis_index("c")
    base = wid * _BPW
    pltpu.sync_copy(idx_hbm.at[wid], idx_v)

    def body(j, _):
        pltpu.async_copy(tab_hbm.at[idx_v.at[j]], rows_v, sem).wait()
        pltpu.sync_copy(rows_v, out_hbm.at[pl.ds(base + j * _CH, _CH)])
        return 0

    lax.fori_loop(0, _NCHUNK, body, 0)


def kernel(words, weight):
    keep = jax.random.bernoulli(
        jax.random.key(42), 1.0 - EMBED_P, (VOCAB, 1)
    ).astype(weight.dtype)
    mask = keep / (1.0 - EMBED_P)
    masked = _masked_table(weight, mask)
    idx = words.astype(jnp.int32).reshape(_NW, _NCHUNK, _CH)
    out = _sc_gather(masked, idx)
    return out.reshape(BATCH, HIST, DIM)


# speedup vs baseline: 3.8418x; 1.4138x over previous
import functools

import jax
import jax.numpy as jnp
from jax import lax
from jax.experimental import pallas as pl
from jax.experimental.pallas import tpu as pltpu
from jax.experimental.pallas import tpu_sc as plsc

VOCAB = 100000
DIM = 64
EMBED_P = 0.1
BATCH = 4096
HIST = 200

_B = BATCH * HIST

_info = plsc.get_sparse_core_info()
_NC = _info.num_cores
_NS = _info.num_subcores
_NW = _NC * _NS
_BPW = _B // _NW
_CH = 128
_NCHUNK = _BPW // _CH
_NBUF = 8
_LOOK = 4

_MROWS = 4000


def _scale_body(w_ref, m_ref, o_ref):
    m_row = m_ref[...].reshape(1, _MROWS)
    m_col = lax.transpose(m_row, (1, 0))
    o_ref[...] = w_ref[...] * m_col


def _masked_table(weight, mask_lanes):
    grid = VOCAB // _MROWS
    return pl.pallas_call(
        _scale_body,
        grid=(grid,),
        in_specs=[
            pl.BlockSpec((_MROWS, DIM), lambda i: (i, 0)),
            pl.BlockSpec((1, 1, _MROWS), lambda i: (i, 0, 0)),
        ],
        out_specs=pl.BlockSpec((_MROWS, DIM), lambda i: (i, 0)),
        out_shape=jax.ShapeDtypeStruct((VOCAB, DIM), jnp.float32),
    )(weight, mask_lanes)


_mesh = plsc.VectorSubcoreMesh(core_axis_name="c", subcore_axis_name="s")


@functools.partial(
    pl.kernel,
    mesh=_mesh,
    out_type=jax.ShapeDtypeStruct((_B, DIM), jnp.float32),
    scratch_types=[
        pltpu.VMEM((_NCHUNK, _CH), jnp.int32),
    ]
    + [pltpu.VMEM((_CH, DIM), jnp.float32) for _ in range(_NBUF)]
    + [pltpu.SemaphoreType.DMA for _ in range(2 * _NBUF)],
    compiler_params=pltpu.CompilerParams(use_tc_tiling_on_sc=False),
)
def _sc_gather(tab_hbm, idx_hbm, out_hbm, idx_v, *bufs_and_sems):
    rows = bufs_and_sems[:_NBUF]
    gsem = bufs_and_sems[_NBUF:2 * _NBUF]
    ssem = bufs_and_sems[2 * _NBUF:]
    wid = lax.axis_index("s") * _NC + lax.axis_index("c")
    base = wid * _BPW
    pltpu.sync_copy(idx_hbm.at[wid], idx_v)

    def start_gather(j, b):
        pltpu.async_copy(tab_hbm.at[idx_v.at[j]], rows[b], gsem[b])

    def wait_gather(j, b):
        pltpu.make_async_copy(tab_hbm.at[idx_v.at[j]], rows[b], gsem[b]).wait()

    def start_scatter(j, b):
        pltpu.async_copy(
            rows[b], out_hbm.at[pl.ds(base + j * _CH, _CH)], ssem[b]
        )

    def wait_scatter(j, b):
        pltpu.make_async_copy(
            rows[b], out_hbm.at[pl.ds(base + j * _CH, _CH)], ssem[b]
        ).wait()

    for b in range(_LOOK):
        start_gather(b, b)

    for b in range(_NBUF):
        wait_gather(b, b)
        start_scatter(b, b)
        if b >= _LOOK:
            wait_scatter(b - _LOOK, b - _LOOK)
        start_gather(b + _LOOK, (b + _LOOK) % _NBUF)

    def round_body(r, _):
        for b in range(_NBUF):
            j = r * _NBUF + b
            wait_gather(j, b)
            start_scatter(j, b)
            wait_scatter(j - _LOOK, (b - _LOOK) % _NBUF)
            start_gather(j + _LOOK, (b + _LOOK) % _NBUF)
        return 0

    lax.fori_loop(1, _NCHUNK // _NBUF - 1, round_body, 0)

    r = _NCHUNK // _NBUF - 1
    for b in range(_NBUF):
        j = r * _NBUF + b
        wait_gather(j, b)
        start_scatter(j, b)
        wait_scatter(j - _LOOK, (b - _LOOK) % _NBUF)
        if j + _LOOK < _NCHUNK:
            start_gather(j + _LOOK, (b + _LOOK) % _NBUF)

    for b in range(_NBUF - _LOOK, _NBUF):
        j = r * _NBUF + b
        wait_scatter(j, b)


def kernel(words, weight):
    keep = jax.random.bernoulli(
        jax.random.key(42), 1.0 - EMBED_P, (VOCAB,)
    ).astype(weight.dtype)
    mask_lanes = (keep / (1.0 - EMBED_P)).reshape(VOCAB // _MROWS, 1, _MROWS)
    masked = _masked_table(weight, mask_lanes)
    idx = words.astype(jnp.int32).reshape(_NW, _NCHUNK, _CH)
    out = _sc_gather(masked, idx)
    return out.reshape(BATCH, HIST, DIM)
